# baseline (device time: 105516 ns/iter reference)
import jax
import jax.numpy as jnp
from jax import lax
from jax.experimental import pallas as pl
from jax.experimental.pallas import tpu as pltpu

N_DEV = 32
N_RING = 16
SUB = 4

_MESH_COORDS = [
    (x, y, z)
    for z in range(4)
    for y in range(4)
    for x in ((0, 1) if y % 2 == 0 else (1, 0))
]
_LOGICAL_OF = {c: k for k, c in enumerate(_MESH_COORDS)}

_HAM16 = [
    (0, 0), (1, 0), (2, 0), (3, 0),
    (3, 1), (3, 2), (3, 3), (2, 3),
    (2, 2), (2, 1), (1, 1), (1, 2),
    (1, 3), (0, 3), (0, 2), (0, 1),
]
LID0 = [_LOGICAL_OF[(0, y, z)] for (y, z) in _HAM16]
LID1 = [_LOGICAL_OF[(1, y, z)] for (y, z) in _HAM16]
EPAIR = [min(a, b) for a, b in zip(LID0, LID1)]
RPOS16 = [0] * N_DEV
for _q in range(N_RING):
    RPOS16[LID0[_q]] = _q
    RPOS16[LID1[_q]] = _q


def kernel(x, w_mat):
    m_tot, k_shard = x.shape
    _, n = w_mat.shape
    m_per = m_tot // N_DEV
    m_pair = 2 * m_per
    nh = n // 2
    ns = nh // SUB

    def body(x_ref, w_ref, lid0_ref, lid1_ref, rpos_ref, out_ref,
             wbf_ref, xr_ref, pr_ref, cw_ref, ccw_ref, px_ref,
             cw_send, cw_recv, ccw_send, ccw_recv, px_send, px_recv):
        my = lax.axis_index("i")

        idx32 = lax.broadcasted_iota(jnp.int32, (1, N_DEV), 1)
        idx16 = lax.broadcasted_iota(jnp.int32, (1, N_RING), 1)

        def lut32(tbl, i):
            return jnp.sum(jnp.where(idx32 == i, tbl, 0))

        def lut16(tbl, i):
            return jnp.sum(jnp.where(idx16 == i, tbl, 0))

        r = lut32(rpos_ref[...], my)
        rem8 = lax.rem(my, 8)
        ypar = lax.rem(lax.div(rem8, 2), 2)
        xpar = lax.rem(rem8, 2)
        xc = jnp.where(ypar == 0, xpar, 1 - xpar)
        partner = my + 1 - 2 * lax.rem(my, 2)

        def layer_dev(q):
            return jnp.where(xc == 0, lut16(lid0_ref[...], q),
                             lut16(lid1_ref[...], q))

        right = layer_dev(lax.rem(r + 1, N_RING))
        left = layer_dev(lax.rem(r - 1 + N_RING, N_RING))

        barrier_sem = pltpu.get_barrier_semaphore()
        for nbr in (left, right, partner):
            pl.semaphore_signal(
                barrier_sem, inc=1,
                device_id=(nbr,), device_id_type=pl.DeviceIdType.MESH,
            )
        pl.semaphore_wait(barrier_sem, 3)

        wbf_ref[...] = w_ref[...].astype(jnp.bfloat16)
        for q in range(N_RING):
            xr_ref[q * m_pair:(q + 1) * m_pair, :] = (
                x_ref[EPAIR[q] * m_per:EPAIR[q] * m_per + m_pair, :]
                .astype(jnp.bfloat16))

        def prow_cw(h):
            return lax.rem(r - h - 2 + 4 * N_RING, N_RING) * m_pair

        def prow_ccw(h):
            return lax.rem(r + h + 2, N_RING) * m_pair

        def mk(dir_ref, send_sems, recv_sems, dst_dev, h, s):
            src_slot = N_RING - 1 if h == 0 else h - 1
            return pltpu.make_async_remote_copy(
                src_ref=dir_ref.at[src_slot, :, s * ns:(s + 1) * ns],
                dst_ref=dir_ref.at[h, :, s * ns:(s + 1) * ns],
                send_sem=send_sems.at[h, s],
                recv_sem=recv_sems.at[h, s],
                device_id=(dst_dev,),
                device_id_type=pl.DeviceIdType.MESH,
            )

        cw_ref[N_RING - 1] = jnp.dot(
            xr_ref[pl.ds(prow_cw(-1), m_pair), :], wbf_ref[:, :nh],
            preferred_element_type=jnp.float32).astype(jnp.bfloat16)
        ccw_ref[N_RING - 1] = jnp.dot(
            xr_ref[pl.ds(prow_ccw(-1), m_pair), :], wbf_ref[:, nh:],
            preferred_element_type=jnp.float32).astype(jnp.bfloat16)
        for s in range(SUB):
            mk(cw_ref, cw_send, cw_recv, right, 0, s).start()
            mk(ccw_ref, ccw_send, ccw_recv, left, 0, s).start()

        pr_ref[...] = jnp.dot(
            xr_ref[...], wbf_ref[...],
            preferred_element_type=jnp.float32).astype(jnp.bfloat16)

        for h in range(N_RING - 1):
            p_cw = pr_ref[pl.ds(prow_cw(h), m_pair), :nh]
            p_ccw = pr_ref[pl.ds(prow_ccw(h), m_pair), nh:]

            for s in range(SUB):
                sl = slice(s * ns, (s + 1) * ns)
                for dir_ref, send_sems, recv_sems, dst, p in (
                    (cw_ref, cw_send, cw_recv, right, p_cw),
                    (ccw_ref, ccw_send, ccw_recv, left, p_ccw),
                ):
                    mk(dir_ref, send_sems, recv_sems, dst, h, s).wait_recv()
                    dir_ref[h, :, sl] = dir_ref[h, :, sl] + p[:, sl]
                    if h < N_RING - 2:
                        mk(dir_ref, send_sems, recv_sems, dst, h + 1, s).start()

        o_me = lax.rem(my, 2) * m_per
        o_pt = (1 - lax.rem(my, 2)) * m_per
        for d, dir_ref in ((0, cw_ref), (1, ccw_ref)):
            rdma = pltpu.make_async_remote_copy(
                src_ref=dir_ref.at[N_RING - 2, pl.ds(o_pt, m_per), :],
                dst_ref=px_ref.at[d],
                send_sem=px_send.at[d],
                recv_sem=px_recv.at[d],
                device_id=(partner,),
                device_id_type=pl.DeviceIdType.MESH,
            )
            rdma.start()

        for d, dir_ref, col0 in ((0, cw_ref, 0), (1, ccw_ref, nh)):
            rdma = pltpu.make_async_remote_copy(
                src_ref=dir_ref.at[N_RING - 2, pl.ds(o_pt, m_per), :],
                dst_ref=px_ref.at[d],
                send_sem=px_send.at[d],
                recv_sem=px_recv.at[d],
                device_id=(partner,),
                device_id_type=pl.DeviceIdType.MESH,
            )
            rdma.wait_recv()
            acc = (dir_ref[N_RING - 2, pl.ds(o_me, m_per), :]
                   .astype(jnp.float32)
                   + px_ref[d].astype(jnp.float32))
            out_ref[:, col0:col0 + nh] = acc * jax.nn.sigmoid(acc)

        for h in range(N_RING - 1):
            for s in range(SUB):
                mk(cw_ref, cw_send, cw_recv, right, h, s).wait_send()
                mk(ccw_ref, ccw_send, ccw_recv, left, h, s).wait_send()
        for d, dir_ref in ((0, cw_ref), (1, ccw_ref)):
            pltpu.make_async_remote_copy(
                src_ref=dir_ref.at[N_RING - 2, pl.ds(o_pt, m_per), :],
                dst_ref=px_ref.at[d],
                send_sem=px_send.at[d],
                recv_sem=px_recv.at[d],
                device_id=(partner,),
                device_id_type=pl.DeviceIdType.MESH,
            ).wait_send()

    return pl.pallas_call(
        body,
        out_shape=jax.ShapeDtypeStruct((m_per, n), jnp.float32),
        in_specs=[pl.BlockSpec(memory_space=pltpu.VMEM)] * 5,
        out_specs=pl.BlockSpec(memory_space=pltpu.VMEM),
        scratch_shapes=[
            pltpu.VMEM((k_shard, n), jnp.bfloat16),
            pltpu.VMEM((m_tot, k_shard), jnp.bfloat16),
            pltpu.VMEM((m_tot, n), jnp.bfloat16),
            pltpu.VMEM((N_RING, m_pair, nh), jnp.bfloat16),
            pltpu.VMEM((N_RING, m_pair, nh), jnp.bfloat16),
            pltpu.VMEM((2, m_per, nh), jnp.bfloat16),
            pltpu.SemaphoreType.DMA((N_RING - 1, SUB)),
            pltpu.SemaphoreType.DMA((N_RING - 1, SUB)),
            pltpu.SemaphoreType.DMA((N_RING - 1, SUB)),
            pltpu.SemaphoreType.DMA((N_RING - 1, SUB)),
            pltpu.SemaphoreType.DMA((2,)),
            pltpu.SemaphoreType.DMA((2,)),
        ],
        compiler_params=pltpu.CompilerParams(
            collective_id=0, vmem_limit_bytes=100 * 1024 * 1024,
        ),
    )(x, w_mat,
      jnp.array([LID0], dtype=jnp.int32),
      jnp.array([LID1], dtype=jnp.int32),
      jnp.array([RPOS16], dtype=jnp.int32))


# device time: 101282 ns/iter; 1.0418x vs baseline; 1.0418x over previous
import jax
import jax.numpy as jnp
from jax import lax
from jax.experimental import pallas as pl
from jax.experimental.pallas import tpu as pltpu

N_DEV = 32
N_RING = 16
SUB = 4

_MESH_COORDS = [
    (x, y, z)
    for z in range(4)
    for y in range(4)
    for x in ((0, 1) if y % 2 == 0 else (1, 0))
]
_LOGICAL_OF = {c: k for k, c in enumerate(_MESH_COORDS)}

_HAM16 = [
    (0, 0), (1, 0), (2, 0), (3, 0),
    (3, 1), (3, 2), (3, 3), (2, 3),
    (2, 2), (2, 1), (1, 1), (1, 2),
    (1, 3), (0, 3), (0, 2), (0, 1),
]
LID0 = [_LOGICAL_OF[(0, y, z)] for (y, z) in _HAM16]
LID1 = [_LOGICAL_OF[(1, y, z)] for (y, z) in _HAM16]
EPAIR = [min(a, b) for a, b in zip(LID0, LID1)]
RPOS16 = [0] * N_DEV
for _q in range(N_RING):
    RPOS16[LID0[_q]] = _q
    RPOS16[LID1[_q]] = _q


def kernel(x, w_mat):
    m_tot, k_shard = x.shape
    _, n = w_mat.shape
    m_per = m_tot // N_DEV
    m_pair = 2 * m_per
    nh = n // 2
    ns = nh // SUB

    def body(x_ref, w_ref, lid0_ref, lid1_ref, rpos_ref, out_ref,
             wbf_ref, pr_ref, cw_ref, ccw_ref, px_ref,
             cw_send, cw_recv, ccw_send, ccw_recv, px_send, px_recv):
        my = lax.axis_index("i")

        idx32 = lax.broadcasted_iota(jnp.int32, (1, N_DEV), 1)
        idx16 = lax.broadcasted_iota(jnp.int32, (1, N_RING), 1)

        def lut32(tbl, i):
            return jnp.sum(jnp.where(idx32 == i, tbl, 0))

        def lut16(tbl, i):
            return jnp.sum(jnp.where(idx16 == i, tbl, 0))

        r = lut32(rpos_ref[...], my)
        rem8 = lax.rem(my, 8)
        ypar = lax.rem(lax.div(rem8, 2), 2)
        xpar = lax.rem(rem8, 2)
        xc = jnp.where(ypar == 0, xpar, 1 - xpar)
        partner = my + 1 - 2 * lax.rem(my, 2)

        def layer_dev(q):
            return jnp.where(xc == 0, lut16(lid0_ref[...], q),
                             lut16(lid1_ref[...], q))

        right = layer_dev(lax.rem(r + 1, N_RING))
        left = layer_dev(lax.rem(r - 1 + N_RING, N_RING))

        barrier_sem = pltpu.get_barrier_semaphore()
        for nbr in (left, right, partner):
            pl.semaphore_signal(
                barrier_sem, inc=1,
                device_id=(nbr,), device_id_type=pl.DeviceIdType.MESH,
            )
        pl.semaphore_wait(barrier_sem, 3)

        wbf_ref[...] = w_ref[...].astype(jnp.bfloat16)

        def xrows(q):
            return (x_ref[EPAIR[q] * m_per:EPAIR[q] * m_per + m_pair, :]
                    .astype(jnp.bfloat16))

        def prow_cw(h):
            return lax.rem(r - h - 2 + 4 * N_RING, N_RING) * m_pair

        def prow_ccw(h):
            return lax.rem(r + h + 2, N_RING) * m_pair

        def mk(dir_ref, send_sems, recv_sems, dst_dev, h, s):
            src_slot = N_RING - 1 if h == 0 else h - 1
            return pltpu.make_async_remote_copy(
                src_ref=dir_ref.at[src_slot, :, s * ns:(s + 1) * ns],
                dst_ref=dir_ref.at[h, :, s * ns:(s + 1) * ns],
                send_sem=send_sems.at[h, s],
                recv_sem=recv_sems.at[h, s],
                device_id=(dst_dev,),
                device_id_type=pl.DeviceIdType.MESH,
            )

        def xrow0(q):
            lid = lut16(lid0_ref[...], q)
            return (lid - lax.rem(lid, 2)) * m_per

        x0_cw = x_ref[pl.ds(xrow0(lax.rem(r - 1 + N_RING, N_RING)),
                            m_pair), :].astype(jnp.bfloat16)
        x0_ccw = x_ref[pl.ds(xrow0(lax.rem(r + 1, N_RING)),
                             m_pair), :].astype(jnp.bfloat16)
        cw_ref[N_RING - 1] = jnp.dot(
            x0_cw, wbf_ref[:, :nh],
            preferred_element_type=jnp.float32).astype(jnp.bfloat16)
        ccw_ref[N_RING - 1] = jnp.dot(
            x0_ccw, wbf_ref[:, nh:],
            preferred_element_type=jnp.float32).astype(jnp.bfloat16)
        for s in range(SUB):
            mk(cw_ref, cw_send, cw_recv, right, 0, s).start()
            mk(ccw_ref, ccw_send, ccw_recv, left, 0, s).start()

        for q in range(N_RING):
            pr_ref[q * m_pair:(q + 1) * m_pair, :] = jnp.dot(
                xrows(q), wbf_ref[...],
                preferred_element_type=jnp.float32).astype(jnp.bfloat16)

        o_me = lax.rem(my, 2) * m_per
        o_pt = (1 - lax.rem(my, 2)) * m_per

        def mk2(dir_ref, d, s):
            sl = pl.ds(s * ns, ns)
            return pltpu.make_async_remote_copy(
                src_ref=dir_ref.at[N_RING - 2, pl.ds(o_pt, m_per), sl],
                dst_ref=px_ref.at[d, :, sl],
                send_sem=px_send.at[d, s],
                recv_sem=px_recv.at[d, s],
                device_id=(partner,),
                device_id_type=pl.DeviceIdType.MESH,
            )

        for h in range(N_RING - 1):
            p_cw = pr_ref[pl.ds(prow_cw(h), m_pair), :nh]
            p_ccw = pr_ref[pl.ds(prow_ccw(h), m_pair), nh:]

            for s in range(SUB):
                sl = slice(s * ns, (s + 1) * ns)
                for d, dir_ref, send_sems, recv_sems, dst, p in (
                    (0, cw_ref, cw_send, cw_recv, right, p_cw),
                    (1, ccw_ref, ccw_send, ccw_recv, left, p_ccw),
                ):
                    mk(dir_ref, send_sems, recv_sems, dst, h, s).wait_recv()
                    dir_ref[h, :, sl] = dir_ref[h, :, sl] + p[:, sl]
                    if h < N_RING - 2:
                        mk(dir_ref, send_sems, recv_sems, dst, h + 1, s).start()
                    else:
                        mk2(dir_ref, d, s).start()

        for s in range(SUB):
            for d, dir_ref, col0 in ((0, cw_ref, 0), (1, ccw_ref, nh)):
                sl = pl.ds(s * ns, ns)
                mk2(dir_ref, d, s).wait_recv()
                acc = (dir_ref[N_RING - 2, pl.ds(o_me, m_per), sl]
                       .astype(jnp.float32)
                       + px_ref[d, :, sl].astype(jnp.float32))
                out_ref[:, col0 + s * ns:col0 + (s + 1) * ns] = (
                    acc * jax.nn.sigmoid(acc))

        for h in range(N_RING - 1):
            for s in range(SUB):
                mk(cw_ref, cw_send, cw_recv, right, h, s).wait_send()
                mk(ccw_ref, ccw_send, ccw_recv, left, h, s).wait_send()
        for s in range(SUB):
            mk2(cw_ref, 0, s).wait_send()
            mk2(ccw_ref, 1, s).wait_send()

    return pl.pallas_call(
        body,
        out_shape=jax.ShapeDtypeStruct((m_per, n), jnp.float32),
        in_specs=[pl.BlockSpec(memory_space=pltpu.VMEM)] * 5,
        out_specs=pl.BlockSpec(memory_space=pltpu.VMEM),
        scratch_shapes=[
            pltpu.VMEM((k_shard, n), jnp.bfloat16),
            pltpu.VMEM((m_tot, n), jnp.bfloat16),
            pltpu.VMEM((N_RING, m_pair, nh), jnp.bfloat16),
            pltpu.VMEM((N_RING, m_pair, nh), jnp.bfloat16),
            pltpu.VMEM((2, m_per, nh), jnp.bfloat16),
            pltpu.SemaphoreType.DMA((N_RING - 1, SUB)),
            pltpu.SemaphoreType.DMA((N_RING - 1, SUB)),
            pltpu.SemaphoreType.DMA((N_RING - 1, SUB)),
            pltpu.SemaphoreType.DMA((N_RING - 1, SUB)),
            pltpu.SemaphoreType.DMA((2, SUB)),
            pltpu.SemaphoreType.DMA((2, SUB)),
        ],
        compiler_params=pltpu.CompilerParams(
            collective_id=0, vmem_limit_bytes=100 * 1024 * 1024,
        ),
    )(x, w_mat,
      jnp.array([LID0], dtype=jnp.int32),
      jnp.array([LID1], dtype=jnp.int32),
      jnp.array([RPOS16], dtype=jnp.int32))
